# baseline (speedup 1.0000x reference)
import jax
import jax.numpy as jnp
from jax.experimental import pallas as pl
from jax.experimental.pallas import tpu as pltpu

D = 2048
N_TOK = 32768
B = 16
R = 2048
NS = 4
H = D // NS
CW = 256
NEG = -1e30


def _pool_body(idx_ref, watt_ref, batt_ref, *refs):
    a_refs = refs[:NS]
    acc_refs = refs[NS:2 * NS]
    l_ref = refs[2 * NS]
    m_ref = refs[2 * NS + 1]
    i = pl.program_id(0)

    @pl.when(i == 0)
    def _init():
        m_ref[...] = jnp.full_like(m_ref, NEG)
        l_ref[...] = jnp.zeros_like(l_ref)
        for acc_ref in acc_refs:
            acc_ref[...] = jnp.zeros_like(acc_ref)

    a = [r[...] for r in a_refs]
    w = watt_ref[...]
    s = batt_ref[0, 0]
    for k in range(NS):
        s = s + jax.lax.dot_general(
            a[k], w[k * H:(k + 1) * H], (((1,), (0,)), ((), ())),
            preferred_element_type=jnp.float32)

    pos = i * R + jax.lax.broadcasted_iota(jnp.int32, (R, B), 0)
    st = idx_ref[...][:, 0][None, :]
    en = idx_ref[...][:, 1][None, :]
    mask = (pos >= st) & (pos <= en)

    sb = jnp.where(mask, s, NEG)
    bm = jnp.max(sb, axis=0)
    m_old = m_ref[0]
    m_new = jnp.maximum(m_old, bm)
    alpha = jnp.exp(m_old - m_new)
    e = jnp.exp(sb - m_new[None, :])
    l_ref[0] = alpha * l_ref[0] + jnp.sum(e, axis=0)
    m_ref[0] = m_new
    for k in range(NS):
        acc_refs[k][...] = acc_refs[k][...] * alpha[:, None] + \
            jax.lax.dot_general(e, a[k], (((0,), (0,)), ((), ())),
                                preferred_element_type=jnp.float32)


def _proj_body(*refs):
    acc_refs = refs[:NS]
    l_ref = refs[NS]
    w_refs = refs[NS + 1:2 * NS + 1]
    bout_ref = refs[2 * NS + 1]
    out_ref = refs[2 * NS + 2]
    out = bout_ref[...]
    for k in range(NS):
        p = acc_refs[k][...] / l_ref[0][:, None]
        out = out + jax.lax.dot_general(
            p, w_refs[k][...], (((1,), (0,)), ((), ())),
            preferred_element_type=jnp.float32)
    out_ref[...] = out


@jax.jit
def kernel(atom_features, index_list, W_att, b_att, W_out, b_out):
    nb = N_TOK // R
    outs = pl.pallas_call(
        _pool_body,
        grid=(nb,),
        in_specs=[
            pl.BlockSpec((B, 2), lambda i: (0, 0)),
            pl.BlockSpec((D, 1), lambda i: (0, 0)),
            pl.BlockSpec((1, 1), lambda i: (0, 0)),
        ] + [
            pl.BlockSpec((R, H), lambda i, k=k: (i, k))
            for k in range(NS)
        ],
        out_specs=[
            pl.BlockSpec((B, H), lambda i: (0, 0)) for _ in range(NS)
        ] + [
            pl.BlockSpec((1, B), lambda i: (0, 0)),
        ],
        out_shape=[
            jax.ShapeDtypeStruct((B, H), jnp.float32) for _ in range(NS)
        ] + [
            jax.ShapeDtypeStruct((1, B), jnp.float32),
        ],
        scratch_shapes=[
            pltpu.VMEM((1, B), jnp.float32),
        ],
    )(index_list.astype(jnp.int32), W_att, b_att.reshape(1, 1),
      *([atom_features] * NS))
    accs, l = outs[:NS], outs[NS]

    return pl.pallas_call(
        _proj_body,
        grid=(D // CW,),
        in_specs=[
            pl.BlockSpec((B, H), lambda j: (0, 0)) for _ in range(NS)
        ] + [
            pl.BlockSpec((1, B), lambda j: (0, 0)),
        ] + [
            pl.BlockSpec((H, CW), lambda j, k=k: (k, j))
            for k in range(NS)
        ] + [
            pl.BlockSpec((1, CW), lambda j: (0, j)),
        ],
        out_specs=pl.BlockSpec((B, CW), lambda j: (0, j)),
        out_shape=jax.ShapeDtypeStruct((B, D), jnp.float32),
    )(*accs, l, *([W_out] * NS), b_out.reshape(1, D))


# speedup vs baseline: 1.1718x; 1.1718x over previous
import jax
import jax.numpy as jnp
from jax.experimental import pallas as pl
from jax.experimental.pallas import tpu as pltpu

D = 2048
N_TOK = 32768
B = 16
R = 2048
NR = 4
RS = R // NR
CW = 256
NEG = -1e30


def _pool_body(idx_ref, watt_ref, batt_ref, *refs):
    a_refs = refs[:NR]
    acc_ref, l_ref, m_ref = refs[NR], refs[NR + 1], refs[NR + 2]
    i = pl.program_id(0)

    @pl.when(i == 0)
    def _init():
        m_ref[...] = jnp.full_like(m_ref, NEG)
        l_ref[...] = jnp.zeros_like(l_ref)
        acc_ref[...] = jnp.zeros_like(acc_ref)

    a = [r[...] for r in a_refs]
    w = watt_ref[...]
    st = idx_ref[...][:, 0][None, :]
    en = idx_ref[...][:, 1][None, :]

    sbs = []
    for k in range(NR):
        s = jax.lax.dot_general(
            a[k], w, (((1,), (0,)), ((), ())),
            preferred_element_type=jnp.float32) + batt_ref[0, 0]
        pos = (i * R + k * RS) + jax.lax.broadcasted_iota(
            jnp.int32, (RS, B), 0)
        mask = (pos >= st) & (pos <= en)
        sbs.append(jnp.where(mask, s, NEG))

    bm = sbs[0].max(axis=0)
    for k in range(1, NR):
        bm = jnp.maximum(bm, sbs[k].max(axis=0))
    m_old = m_ref[0]
    m_new = jnp.maximum(m_old, bm)
    alpha = jnp.exp(m_old - m_new)
    es = [jnp.exp(sb - m_new[None, :]) for sb in sbs]
    lsum = es[0].sum(axis=0)
    for k in range(1, NR):
        lsum = lsum + es[k].sum(axis=0)
    l_ref[0] = alpha * l_ref[0] + lsum
    m_ref[0] = m_new
    upd = jax.lax.dot_general(es[0], a[0], (((0,), (0,)), ((), ())),
                              preferred_element_type=jnp.float32)
    for k in range(1, NR):
        upd = upd + jax.lax.dot_general(
            es[k], a[k], (((0,), (0,)), ((), ())),
            preferred_element_type=jnp.float32)
    acc_ref[...] = acc_ref[...] * alpha[:, None] + upd


def _proj_body(acc_ref, l_ref, wout_ref, bout_ref, out_ref):
    pooled = acc_ref[...] / l_ref[0][:, None]
    out_ref[...] = jax.lax.dot_general(
        pooled, wout_ref[...], (((1,), (0,)), ((), ())),
        preferred_element_type=jnp.float32) + bout_ref[...]


@jax.jit
def kernel(atom_features, index_list, W_att, b_att, W_out, b_out):
    nb = N_TOK // R
    acc, l = pl.pallas_call(
        _pool_body,
        grid=(nb,),
        in_specs=[
            pl.BlockSpec((B, 2), lambda i: (0, 0)),
            pl.BlockSpec((D, 1), lambda i: (0, 0)),
            pl.BlockSpec((1, 1), lambda i: (0, 0)),
        ] + [
            pl.BlockSpec((RS, D), lambda i, k=k: (i * NR + k, 0))
            for k in range(NR)
        ],
        out_specs=[
            pl.BlockSpec((B, D), lambda i: (0, 0)),
            pl.BlockSpec((1, B), lambda i: (0, 0)),
        ],
        out_shape=[
            jax.ShapeDtypeStruct((B, D), jnp.float32),
            jax.ShapeDtypeStruct((1, B), jnp.float32),
        ],
        scratch_shapes=[
            pltpu.VMEM((1, B), jnp.float32),
        ],
    )(index_list.astype(jnp.int32), W_att, b_att.reshape(1, 1),
      *([atom_features] * NR))

    return pl.pallas_call(
        _proj_body,
        grid=(D // CW,),
        in_specs=[
            pl.BlockSpec((B, D), lambda j: (0, 0)),
            pl.BlockSpec((1, B), lambda j: (0, 0)),
            pl.BlockSpec((D, CW), lambda j: (0, j)),
            pl.BlockSpec((1, CW), lambda j: (0, j)),
        ],
        out_specs=pl.BlockSpec((B, CW), lambda j: (0, j)),
        out_shape=jax.ShapeDtypeStruct((B, D), jnp.float32),
    )(acc, l, W_out, b_out.reshape(1, D))


# speedup vs baseline: 1.1790x; 1.0061x over previous
import jax
import jax.numpy as jnp
from jax.experimental import pallas as pl
from jax.experimental.pallas import tpu as pltpu

D = 2048
N_TOK = 32768
B = 16
NC = 2
R = 2048
NR = 2
RS = R // NR
NB2 = N_TOK // R // NC
CW = 256
NEG = -1e30


def _pool_body(idx_ref, watt_ref, batt_ref, *refs):
    a_refs = refs[:NR]
    acc_ref, l_ref, m_ref = refs[NR], refs[NR + 1], refs[NR + 2]
    c = pl.program_id(0)
    i = pl.program_id(1)

    @pl.when(i == 0)
    def _init():
        m_ref[...] = jnp.full_like(m_ref, NEG)
        l_ref[...] = jnp.zeros_like(l_ref)
        acc_ref[...] = jnp.zeros_like(acc_ref)

    a = [r[...] for r in a_refs]
    w = watt_ref[...]
    st = idx_ref[...][:, 0][None, :]
    en = idx_ref[...][:, 1][None, :]

    base = (c * NB2 + i) * R
    sbs = []
    for k in range(NR):
        s = jax.lax.dot_general(
            a[k], w, (((1,), (0,)), ((), ())),
            preferred_element_type=jnp.float32) + batt_ref[0, 0]
        pos = (base + k * RS) + jax.lax.broadcasted_iota(
            jnp.int32, (RS, B), 0)
        mask = (pos >= st) & (pos <= en)
        sbs.append(jnp.where(mask, s, NEG))

    bm = sbs[0].max(axis=0)
    for k in range(1, NR):
        bm = jnp.maximum(bm, sbs[k].max(axis=0))
    m_old = m_ref[0, 0]
    m_new = jnp.maximum(m_old, bm)
    alpha = jnp.exp(m_old - m_new)
    es = [jnp.exp(sb - m_new[None, :]) for sb in sbs]
    lsum = es[0].sum(axis=0)
    for k in range(1, NR):
        lsum = lsum + es[k].sum(axis=0)
    l_ref[0, 0] = alpha * l_ref[0, 0] + lsum
    m_ref[0, 0] = m_new
    upd = jax.lax.dot_general(es[0], a[0], (((0,), (0,)), ((), ())),
                              preferred_element_type=jnp.float32)
    for k in range(1, NR):
        upd = upd + jax.lax.dot_general(
            es[k], a[k], (((0,), (0,)), ((), ())),
            preferred_element_type=jnp.float32)
    acc_ref[0] = acc_ref[0] * alpha[:, None] + upd


def _proj_body(acc_ref, l_ref, m_ref, wout_ref, bout_ref, out_ref):
    m = m_ref[...][:, 0, :]
    l = l_ref[...][:, 0, :]
    m_star = jnp.max(m, axis=0)
    wgt = jnp.exp(m - m_star[None, :])
    denom = jnp.sum(l * wgt, axis=0)
    coef = wgt / denom[None, :]
    pooled = acc_ref[0] * coef[0][:, None]
    for c in range(1, NC):
        pooled = pooled + acc_ref[c] * coef[c][:, None]
    out_ref[...] = jax.lax.dot_general(
        pooled, wout_ref[...], (((1,), (0,)), ((), ())),
        preferred_element_type=jnp.float32) + bout_ref[...]


@jax.jit
def kernel(atom_features, index_list, W_att, b_att, W_out, b_out):
    acc, l, m = pl.pallas_call(
        _pool_body,
        grid=(NC, NB2),
        in_specs=[
            pl.BlockSpec((B, 2), lambda c, i: (0, 0)),
            pl.BlockSpec((D, 1), lambda c, i: (0, 0)),
            pl.BlockSpec((1, 1), lambda c, i: (0, 0)),
        ] + [
            pl.BlockSpec((RS, D),
                         lambda c, i, k=k: ((c * NB2 + i) * NR + k, 0))
            for k in range(NR)
        ],
        out_specs=[
            pl.BlockSpec((1, B, D), lambda c, i: (c, 0, 0)),
            pl.BlockSpec((1, 1, B), lambda c, i: (c, 0, 0)),
            pl.BlockSpec((1, 1, B), lambda c, i: (c, 0, 0)),
        ],
        out_shape=[
            jax.ShapeDtypeStruct((NC, B, D), jnp.float32),
            jax.ShapeDtypeStruct((NC, 1, B), jnp.float32),
            jax.ShapeDtypeStruct((NC, 1, B), jnp.float32),
        ],
        compiler_params=pltpu.CompilerParams(
            dimension_semantics=("parallel", "arbitrary")),
    )(index_list.astype(jnp.int32), W_att, b_att.reshape(1, 1),
      *([atom_features] * NR))

    return pl.pallas_call(
        _proj_body,
        grid=(D // CW,),
        in_specs=[
            pl.BlockSpec((NC, B, D), lambda j: (0, 0, 0)),
            pl.BlockSpec((NC, 1, B), lambda j: (0, 0, 0)),
            pl.BlockSpec((NC, 1, B), lambda j: (0, 0, 0)),
            pl.BlockSpec((D, CW), lambda j: (0, j)),
            pl.BlockSpec((1, CW), lambda j: (0, j)),
        ],
        out_specs=pl.BlockSpec((B, CW), lambda j: (0, j)),
        out_shape=jax.ShapeDtypeStruct((B, D), jnp.float32),
    )(acc, l, m, W_out, b_out.reshape(1, D))


# speedup vs baseline: 1.2582x; 1.0672x over previous
import jax
import jax.numpy as jnp
from jax.experimental import pallas as pl
from jax.experimental.pallas import tpu as pltpu

D = 2048
N_TOK = 32768
B = 16
R = 2048
NR = 2
RS = R // NR
CW = 256
NEG = -1e30


def _first_sub(idx_ref):
    m = idx_ref[0, 0]
    for b in range(1, B):
        m = jnp.minimum(m, idx_ref[b, 0])
    return m // RS


def _last_sub(idx_ref):
    m = idx_ref[0, 1]
    for b in range(1, B):
        m = jnp.maximum(m, idx_ref[b, 1])
    return m // RS


def _pool_body(sidx_ref, idx_ref, watt_ref, batt_ref, *refs):
    a_refs = refs[:NR]
    acc_ref, l_ref, m_ref = refs[NR], refs[NR + 1], refs[NR + 2]
    i = pl.program_id(0)
    b_lo = _first_sub(sidx_ref)
    b_hi = _last_sub(sidx_ref)

    @pl.when(i == 0)
    def _init():
        m_ref[...] = jnp.full_like(m_ref, NEG)
        l_ref[...] = jnp.zeros_like(l_ref)
        acc_ref[...] = jnp.zeros_like(acc_ref)

    @pl.when(b_lo + i * NR <= b_hi)
    def _step():
        a = [r[...] for r in a_refs]
        w = watt_ref[...]
        st = idx_ref[...][:, 0][None, :]
        en = idx_ref[...][:, 1][None, :]

        sbs = []
        for k in range(NR):
            s = jax.lax.dot_general(
                a[k], w, (((1,), (0,)), ((), ())),
                preferred_element_type=jnp.float32) + batt_ref[0, 0]
            pos = (b_lo + i * NR + k) * RS + jax.lax.broadcasted_iota(
                jnp.int32, (RS, B), 0)
            mask = (pos >= st) & (pos <= en)
            sbs.append(jnp.where(mask, s, NEG))

        bm = sbs[0].max(axis=0)
        for k in range(1, NR):
            bm = jnp.maximum(bm, sbs[k].max(axis=0))
        m_old = m_ref[0]
        m_new = jnp.maximum(m_old, bm)
        alpha = jnp.exp(m_old - m_new)
        es = [jnp.exp(sb - m_new[None, :]) for sb in sbs]
        lsum = es[0].sum(axis=0)
        for k in range(1, NR):
            lsum = lsum + es[k].sum(axis=0)
        l_ref[0] = alpha * l_ref[0] + lsum
        m_ref[0] = m_new
        upd = jax.lax.dot_general(es[0], a[0], (((0,), (0,)), ((), ())),
                                  preferred_element_type=jnp.float32)
        for k in range(1, NR):
            upd = upd + jax.lax.dot_general(
                es[k], a[k], (((0,), (0,)), ((), ())),
                preferred_element_type=jnp.float32)
        acc_ref[...] = acc_ref[...] * alpha[:, None] + upd


def _proj_body(acc_ref, l_ref, wout_ref, bout_ref, out_ref):
    pooled = acc_ref[...] / l_ref[0][:, None]
    out_ref[...] = jax.lax.dot_general(
        pooled, wout_ref[...], (((1,), (0,)), ((), ())),
        preferred_element_type=jnp.float32) + bout_ref[...]


def _a_spec(k):
    def imap(i, sidx_ref):
        v = _first_sub(sidx_ref) + i * NR + k
        return (jnp.minimum(v, _last_sub(sidx_ref)), 0)
    return pl.BlockSpec((RS, D), imap)


@jax.jit
def kernel(atom_features, index_list, W_att, b_att, W_out, b_out):
    nb = N_TOK // R
    idx32 = index_list.astype(jnp.int32)
    acc, l = pl.pallas_call(
        _pool_body,
        grid_spec=pltpu.PrefetchScalarGridSpec(
            num_scalar_prefetch=1,
            grid=(nb,),
            in_specs=[
                pl.BlockSpec((B, 2), lambda i, s: (0, 0)),
                pl.BlockSpec((D, 1), lambda i, s: (0, 0)),
                pl.BlockSpec((1, 1), lambda i, s: (0, 0)),
            ] + [_a_spec(k) for k in range(NR)],
            out_specs=[
                pl.BlockSpec((B, D), lambda i, s: (0, 0)),
                pl.BlockSpec((1, B), lambda i, s: (0, 0)),
            ],
            scratch_shapes=[
                pltpu.VMEM((1, B), jnp.float32),
            ],
        ),
        out_shape=[
            jax.ShapeDtypeStruct((B, D), jnp.float32),
            jax.ShapeDtypeStruct((1, B), jnp.float32),
        ],
    )(idx32, idx32, W_att, b_att.reshape(1, 1),
      *([atom_features] * NR))

    return pl.pallas_call(
        _proj_body,
        grid=(D // CW,),
        in_specs=[
            pl.BlockSpec((B, D), lambda j: (0, 0)),
            pl.BlockSpec((1, B), lambda j: (0, 0)),
            pl.BlockSpec((D, CW), lambda j: (0, j)),
            pl.BlockSpec((1, CW), lambda j: (0, j)),
        ],
        out_specs=pl.BlockSpec((B, CW), lambda j: (0, j)),
        out_shape=jax.ShapeDtypeStruct((B, D), jnp.float32),
    )(acc, l, W_out, b_out.reshape(1, D))
